# baseline (speedup 1.0000x reference)
import functools

import jax
import jax.numpy as jnp
from jax import lax
from jax.experimental import pallas as pl
from jax.experimental.pallas import tpu as pltpu
from jax.experimental.pallas import tpu_sc as plsc

N = 100000
E = 1600000
HID = 64
NG = 128
XP = 100352
EP = 1638400
DUMMY = XP - 1
NBLK = 49
BLK = 2048
EROWS = EP // 128
ACC_PT = XP // 16

f32 = jnp.float32
i32 = jnp.int32


CHUNK = 640


def _zero_acc(zero_v, acc, lo):
    for k in range(49):
        pltpu.sync_copy(zero_v, acc.at[pl.ds(lo + k * 128, 128)])


def _edge_loop(nchunks, src_fn, dst_fn, table, i_s, i_d, rws, acc,
               isem, gsem, ssem):
    n = nchunks

    def fire_idx(k, b):
        pltpu.async_copy(src_fn(k * CHUNK), i_s[b], isem[b])
        pltpu.async_copy(dst_fn(k * CHUNK), i_d[b], isem[b])

    def wait_idx(k, b):
        pltpu.make_async_copy(src_fn(k * CHUNK), i_s[b], isem[b]).wait()
        pltpu.make_async_copy(dst_fn(k * CHUNK), i_d[b], isem[b]).wait()

    def fire_gather(b):
        pltpu.async_copy(table.at[i_s[b]], rws[b], gsem[b])

    def wait_gather(b):
        pltpu.make_async_copy(table.at[i_s[b]], rws[b], gsem[b]).wait()

    def fire_scatter(b):
        pltpu.async_copy(rws[b], acc.at[i_d[b]], ssem[b], add=True)

    def wait_scatter(b):
        pltpu.make_async_copy(rws[b], acc.at[i_d[b]], ssem[b]).wait()

    fire_idx(0, 0)
    wait_idx(0, 0)
    fire_gather(0)
    if n > 1:
        fire_idx(1, 1)

    q = (n - 2) // 2 if n >= 4 else 0

    def pair(j, _):
        e = 2 * j
        wait_gather(0)
        fire_scatter(0)
        wait_idx(e + 1, 1)
        fire_gather(1)
        wait_scatter(0)
        fire_idx(e + 2, 0)
        wait_gather(1)
        fire_scatter(1)
        wait_idx(e + 2, 0)
        fire_gather(0)
        wait_scatter(1)
        fire_idx(e + 3, 1)
        return 0

    if q > 0:
        lax.fori_loop(0, q, pair, 0)
    for k in range(2 * q, n):
        b = k % 2
        wait_gather(b)
        fire_scatter(b)
        if k + 1 < n:
            wait_idx(k + 1, b ^ 1)
            fire_gather(b ^ 1)
        wait_scatter(b)
        if k + 2 < n:
            fire_idx(k + 2, b)


def _sc_agg1_body(table, srcx, dstx, zeros_hbm, out,
                  i_s0, i_s1, i_d0, i_d1, r0, r1, zero_v, acc,
                  isem0, isem1, gsem0, gsem1, ssem0, ssem1):
    c = lax.axis_index("c")
    s = lax.axis_index("s")
    pltpu.sync_copy(zeros_hbm, zero_v)
    lo = s * ACC_PT
    _zero_acc(zero_v, acc, lo)
    plsc.subcore_barrier()
    e0 = c * (EP // 2) + s * (EP // 32)
    _edge_loop(EP // 32 // CHUNK,
               lambda r: srcx.at[pl.ds(e0 + r, CHUNK)],
               lambda r: dstx.at[pl.ds(e0 + r, CHUNK)],
               table, (i_s0, i_s1), (i_d0, i_d1), (r0, r1), acc,
               (isem0, isem1), (gsem0, gsem1), (ssem0, ssem1))
    plsc.subcore_barrier()
    pltpu.sync_copy(acc.at[pl.ds(lo, ACC_PT)], out.at[c, pl.ds(lo, ACC_PT)])


def _sc_agg4_body(table, srcx, dstx, zeros_hbm, out,
                  i_s0, i_s1, i_d0, i_d1, r0, r1, zero_v, acc,
                  isem0, isem1, gsem0, gsem1, ssem0, ssem1):
    c = lax.axis_index("c")
    s = lax.axis_index("s")
    pltpu.sync_copy(zeros_hbm, zero_v)
    lo = s * ACC_PT
    e0 = s * (EP // 16)
    for rnd in range(2):
        g = 2 * rnd + c
        _zero_acc(zero_v, acc, lo)
        plsc.subcore_barrier()
        _edge_loop(EP // 16 // CHUNK,
                   lambda r: srcx.at[g, pl.ds(e0 + r, CHUNK)],
                   lambda r: dstx.at[pl.ds(e0 + r, CHUNK)],
                   table, (i_s0, i_s1), (i_d0, i_d1), (r0, r1), acc,
                   (isem0, isem1), (gsem0, gsem1), (ssem0, ssem1))
        plsc.subcore_barrier()
        pltpu.sync_copy(acc.at[pl.ds(lo, ACC_PT)],
                        out.at[pl.ds(lo, ACC_PT), pl.ds(g * 16, 16)])
        plsc.subcore_barrier()


_SC_SCRATCH = [
    pltpu.VMEM((CHUNK,), i32),
    pltpu.VMEM((CHUNK,), i32),
    pltpu.VMEM((CHUNK,), i32),
    pltpu.VMEM((CHUNK,), i32),
    pltpu.VMEM((CHUNK, 16), f32),
    pltpu.VMEM((CHUNK, 16), f32),
    pltpu.VMEM((128, 16), f32),
    pltpu.VMEM_SHARED((XP, 16), f32),
    pltpu.SemaphoreType.DMA,
    pltpu.SemaphoreType.DMA,
    pltpu.SemaphoreType.DMA,
    pltpu.SemaphoreType.DMA,
    pltpu.SemaphoreType.DMA,
    pltpu.SemaphoreType.DMA,
]


def _make_sc_kernels():
    mesh = plsc.VectorSubcoreMesh(core_axis_name="c", subcore_axis_name="s")
    params = pltpu.CompilerParams(use_tc_tiling_on_sc=False)
    agg1 = pl.kernel(
        _sc_agg1_body,
        out_type=jax.ShapeDtypeStruct((2, XP, 16), f32),
        mesh=mesh,
        scratch_types=_SC_SCRATCH,
        compiler_params=params,
        name="sc_gin_agg1",
    )
    agg4 = pl.kernel(
        _sc_agg4_body,
        out_type=jax.ShapeDtypeStruct((XP, HID), f32),
        mesh=mesh,
        scratch_types=_SC_SCRATCH,
        compiler_params=params,
        name="sc_gin_agg4",
    )
    return agg1, agg4


def _row_mask(b):
    rows = lax.broadcasted_iota(i32, (BLK, 1), 0) + b * BLK
    return rows < N


def _a1_body(x_ref, aa_ref, ab_ref, w1_ref, b1_ref, y_ref, st_ref):
    b = pl.program_id(0)
    u = x_ref[...] + aa_ref[0] + ab_ref[0]
    y = jnp.dot(u, w1_ref[...], preferred_element_type=f32, precision=lax.Precision.HIGHEST) + b1_ref[...]
    y_ref[...] = y
    @pl.when(b == 0)
    def _():
        st_ref[...] = jnp.zeros_like(st_ref)

    @pl.when(b < NBLK - 1)
    def _():
        st_ref[0:1, :] += jnp.sum(y, axis=0, keepdims=True)
        st_ref[1:2, :] += jnp.sum(y * y, axis=0, keepdims=True)

    @pl.when(b == NBLK - 1)
    def _():
        ym = jnp.where(_row_mask(b), y, 0.0)
        st_ref[0:1, :] += jnp.sum(ym, axis=0, keepdims=True)
        st_ref[1:2, :] += jnp.sum(ym * ym, axis=0, keepdims=True)


def _a4_body(h4_ref, agg_ref, w1_ref, b1_ref, y_ref, st_ref):
    b = pl.program_id(0)
    h = jnp.concatenate([h4_ref[g] for g in range(4)], axis=1)
    u = h + agg_ref[...]
    y = jnp.dot(u, w1_ref[...], preferred_element_type=f32,
                precision=lax.Precision.HIGHEST) + b1_ref[...]
    y_ref[...] = y
    @pl.when(b == 0)
    def _():
        st_ref[...] = jnp.zeros_like(st_ref)

    @pl.when(b < NBLK - 1)
    def _():
        st_ref[0:1, :] += jnp.sum(y, axis=0, keepdims=True)
        st_ref[1:2, :] += jnp.sum(y * y, axis=0, keepdims=True)

    @pl.when(b == NBLK - 1)
    def _():
        ym = jnp.where(_row_mask(b), y, 0.0)
        st_ref[0:1, :] += jnp.sum(ym, axis=0, keepdims=True)
        st_ref[1:2, :] += jnp.sum(ym * ym, axis=0, keepdims=True)


def _b_body(y_ref, st_ref, gamma_ref, beta_ref, w2_ref, b2_ref, h4_ref):
    inv_n = 1.0 / N
    mean = st_ref[0:1, :] * inv_n
    var = st_ref[1:2, :] * inv_n - mean * mean
    scale = gamma_ref[...] * lax.rsqrt(var + 1e-5)
    shift = beta_ref[...] - mean * scale
    h = jnp.maximum(y_ref[...] * scale + shift, 0.0)
    h = jnp.dot(h, w2_ref[...], preferred_element_type=f32, precision=lax.Precision.HIGHEST) + b2_ref[...]
    h = jnp.maximum(h, 0.0)
    for g in range(4):
        h4_ref[g] = h[:, g * 16:(g + 1) * 16]


def _pool_body(h4_ref, ids_ref, w1_ref, b1_ref, w2_ref, b2_ref, out_ref,
               sums, maxs, cnts):
    b = pl.program_id(0)

    @pl.when(b == 0)
    def _():
        sums[...] = jnp.zeros_like(sums)
        cnts[...] = jnp.zeros_like(cnts)
        maxs[...] = jnp.full_like(maxs, -1e30)

    ids = ids_ref[...]
    h = jnp.concatenate([h4_ref[g] for g in range(4)], axis=1)
    valid = ids >= 0
    gid = lax.broadcasted_iota(i32, (BLK, NG), 1)
    oh = jnp.where(valid & (ids == gid), 1.0, 0.0)
    ones_col = jnp.ones((BLK, 1), f32)
    dn = (((0,), (0,)), ((), ()))
    sums[...] += lax.dot_general(oh, h, dn, preferred_element_type=f32, precision=lax.Precision.HIGHEST)
    cnts[...] += lax.dot_general(oh, ones_col, dn, preferred_element_type=f32, precision=lax.Precision.HIGHEST)

    m = jnp.where(valid, h, -1e30)
    s = 1
    while s < BLK:
        ids_sh = jnp.concatenate(
            [ids[s:], jnp.full((s, 1), -2, i32)], axis=0)
        m_sh = jnp.concatenate(
            [m[s:], jnp.full((s, HID), -1e30, f32)], axis=0)
        m = jnp.where(ids_sh == ids, jnp.maximum(m, m_sh), m)
        s *= 2
    prev = jnp.concatenate([jnp.full((1, 1), -3, i32), ids[:-1]], axis=0)
    rs = jnp.where((ids != prev) & valid, 1.0, 0.0)
    ohm = oh * rs
    contrib = lax.dot_general(ohm, m, dn, preferred_element_type=f32, precision=lax.Precision.HIGHEST)
    pres = lax.dot_general(ohm, ones_col, dn, preferred_element_type=f32, precision=lax.Precision.HIGHEST)
    maxs[...] = jnp.maximum(maxs[...],
                            jnp.where(pres > 0, contrib, -1e30))

    @pl.when(b == NBLK - 1)
    def _():
        cnt = cnts[...]
        mean = sums[...] / jnp.maximum(cnt, 1.0)
        maxp = jnp.where(cnt > 0, maxs[...], 0.0)
        feat = jnp.concatenate([mean, maxp], axis=1)
        z = jnp.maximum(
            jnp.dot(feat, w1_ref[...], preferred_element_type=f32, precision=lax.Precision.HIGHEST)
            + b1_ref[...], 0.0)
        out_ref[...] = (jnp.dot(z, w2_ref[...], preferred_element_type=f32, precision=lax.Precision.HIGHEST)
                        + b2_ref[...])


_TC_PARAMS = pltpu.CompilerParams(dimension_semantics=("arbitrary",))


def _full(shape):
    return pl.BlockSpec(shape, lambda b: tuple(0 for _ in shape))


def _tc_a1(x, agg, w1p, b1):
    return pl.pallas_call(
        _a1_body,
        grid=(NBLK,),
        in_specs=[
            pl.BlockSpec((BLK, 16), lambda b: (b, 0)),
            pl.BlockSpec((1, BLK, 16), lambda b: (0, b, 0)),
            pl.BlockSpec((1, BLK, 16), lambda b: (1, b, 0)),
            _full((16, HID)),
            _full((1, HID)),
        ],
        out_specs=(pl.BlockSpec((BLK, HID), lambda b: (b, 0)),
                   _full((2, HID))),
        out_shape=(jax.ShapeDtypeStruct((XP, HID), f32),
                   jax.ShapeDtypeStruct((2, HID), f32)),
        compiler_params=_TC_PARAMS,
        name="tc_gin_a1",
    )(x, agg, agg, w1p, b1)


def _tc_a4(h4, agg, w1, b1):
    return pl.pallas_call(
        _a4_body,
        grid=(NBLK,),
        in_specs=[
            pl.BlockSpec((4, BLK, 16), lambda b: (0, b, 0)),
            pl.BlockSpec((BLK, HID), lambda b: (b, 0)),
            _full((HID, HID)),
            _full((1, HID)),
        ],
        out_specs=(pl.BlockSpec((BLK, HID), lambda b: (b, 0)),
                   _full((2, HID))),
        out_shape=(jax.ShapeDtypeStruct((XP, HID), f32),
                   jax.ShapeDtypeStruct((2, HID), f32)),
        compiler_params=_TC_PARAMS,
        name="tc_gin_a4",
    )(h4, agg, w1, b1)


def _tc_b(y, st, gamma, beta, w2, b2):
    return pl.pallas_call(
        _b_body,
        grid=(NBLK,),
        in_specs=[
            pl.BlockSpec((BLK, HID), lambda b: (b, 0)),
            _full((2, HID)),
            _full((1, HID)),
            _full((1, HID)),
            _full((HID, HID)),
            _full((1, HID)),
        ],
        out_specs=pl.BlockSpec((4, BLK, 16), lambda b: (0, b, 0)),
        out_shape=jax.ShapeDtypeStruct((4, XP, 16), f32),
        compiler_params=_TC_PARAMS,
        name="tc_gin_b",
    )(y, st, gamma, beta, w2, b2)


def _tc_pool(h4, ids, w1, b1, w2, b2):
    return pl.pallas_call(
        _pool_body,
        grid=(NBLK,),
        in_specs=[
            pl.BlockSpec((4, BLK, 16), lambda b: (0, b, 0)),
            pl.BlockSpec((BLK, 1), lambda b: (b, 0)),
            _full((2 * HID, HID)),
            _full((1, HID)),
            _full((HID, 2)),
            _full((1, 2)),
        ],
        out_specs=_full((NG, 2)),
        out_shape=jax.ShapeDtypeStruct((NG, 2), f32),
        scratch_shapes=[
            pltpu.VMEM((NG, HID), f32),
            pltpu.VMEM((NG, HID), f32),
            pltpu.VMEM((NG, 1), f32),
        ],
        compiler_params=_TC_PARAMS,
        name="tc_gin_pool",
    )(h4, ids, w1, b1, w2, b2)


def kernel(x, edge_index, batch,
           conv1_W1, conv1_b1, conv1_gamma, conv1_beta, conv1_W2, conv1_b2,
           conv2_W1, conv2_b1, conv2_gamma, conv2_beta, conv2_W2, conv2_b2,
           conv3_W1, conv3_b1, conv3_gamma, conv3_beta, conv3_W2, conv3_b2,
           clf_W1, clf_b1, clf_W2, clf_b2):
    sc_agg1, sc_agg4 = _make_sc_kernels()

    src = edge_index[0]
    dst = edge_index[1]
    srcp = jnp.pad(src, (0, EP - E))
    dstp = jnp.pad(dst, (0, EP - E), constant_values=DUMMY)
    srcx1 = srcp
    srcx4 = srcp[None, :] + (jnp.arange(4, dtype=i32) * XP)[:, None]
    dstx = dstp

    x_pad = jnp.pad(x, ((0, XP - N), (0, 16 - x.shape[1])))
    ids = jnp.pad(batch, (0, XP - N), constant_values=-1).reshape(XP, 1)
    w1p = jnp.pad(conv1_W1, ((0, 16 - conv1_W1.shape[0]), (0, 0)))

    def row(v):
        return v.reshape(1, -1)

    zeros_hbm = jnp.zeros((128, 16), f32)
    u1 = sc_agg1(x_pad, srcx1, dstx, zeros_hbm)
    y1, st1 = _tc_a1(x_pad, u1, w1p, row(conv1_b1))
    h1 = _tc_b(y1, st1, row(conv1_gamma), row(conv1_beta), conv1_W2,
               row(conv1_b2))

    a2 = sc_agg4(h1.reshape(4 * XP, 16), srcx4, dstx, zeros_hbm)
    y2, st2 = _tc_a4(h1, a2, conv2_W1, row(conv2_b1))
    h2 = _tc_b(y2, st2, row(conv2_gamma), row(conv2_beta), conv2_W2,
               row(conv2_b2))

    a3 = sc_agg4(h2.reshape(4 * XP, 16), srcx4, dstx, zeros_hbm)
    y3, st3 = _tc_a4(h2, a3, conv3_W1, row(conv3_b1))
    h3 = _tc_b(y3, st3, row(conv3_gamma), row(conv3_beta), conv3_W2,
               row(conv3_b2))

    return _tc_pool(h3, ids, clf_W1, row(clf_b1), clf_W2, row(clf_b2))


# speedup vs baseline: 1.0028x; 1.0028x over previous
import functools

import jax
import jax.numpy as jnp
from jax import lax
from jax.experimental import pallas as pl
from jax.experimental.pallas import tpu as pltpu
from jax.experimental.pallas import tpu_sc as plsc

N = 100000
E = 1600000
HID = 64
NG = 128
XP = 100352
EP = 1638400
DUMMY = XP - 1
NBLK = 49
BLK = 2048
EROWS = EP // 128
ACC_PT = XP // 16

f32 = jnp.float32
i32 = jnp.int32


CHUNK = 640


def _zero_acc(zero_v, acc, lo, sem):
    for k in range(49):
        pltpu.async_copy(zero_v, acc.at[pl.ds(lo + k * 128, 128)], sem)
    for k in range(49):
        pltpu.make_async_copy(zero_v, acc.at[pl.ds(lo + k * 128, 128)],
                              sem).wait()


def _edge_loop(nchunks, src_fn, dst_fn, table, i_s, i_d, rws, acc,
               isem, gsem, ssem):
    n = nchunks

    def fire_idx(k, b):
        pltpu.async_copy(src_fn(k * CHUNK), i_s[b], isem[b])
        pltpu.async_copy(dst_fn(k * CHUNK), i_d[b], isem[b])

    def wait_idx(k, b):
        pltpu.make_async_copy(src_fn(k * CHUNK), i_s[b], isem[b]).wait()
        pltpu.make_async_copy(dst_fn(k * CHUNK), i_d[b], isem[b]).wait()

    def fire_gather(b):
        pltpu.async_copy(table.at[i_s[b]], rws[b], gsem[b])

    def wait_gather(b):
        pltpu.make_async_copy(table.at[i_s[b]], rws[b], gsem[b]).wait()

    def fire_scatter(b):
        pltpu.async_copy(rws[b], acc.at[i_d[b]], ssem[b], add=True)

    def wait_scatter(b):
        pltpu.make_async_copy(rws[b], acc.at[i_d[b]], ssem[b]).wait()

    fire_idx(0, 0)
    wait_idx(0, 0)
    fire_gather(0)
    if n > 1:
        fire_idx(1, 1)

    q = (n - 2) // 2 if n >= 4 else 0

    def pair(j, _):
        e = 2 * j
        wait_gather(0)
        fire_scatter(0)
        wait_idx(e + 1, 1)
        fire_gather(1)
        wait_scatter(0)
        fire_idx(e + 2, 0)
        wait_gather(1)
        fire_scatter(1)
        wait_idx(e + 2, 0)
        fire_gather(0)
        wait_scatter(1)
        fire_idx(e + 3, 1)
        return 0

    if q > 0:
        lax.fori_loop(0, q, pair, 0)
    for k in range(2 * q, n):
        b = k % 2
        wait_gather(b)
        fire_scatter(b)
        if k + 1 < n:
            wait_idx(k + 1, b ^ 1)
            fire_gather(b ^ 1)
        wait_scatter(b)
        if k + 2 < n:
            fire_idx(k + 2, b)


def _sc_agg1_body(table, srcx, dstx, zeros_hbm, out,
                  i_s0, i_s1, i_d0, i_d1, r0, r1, zero_v, acc,
                  isem0, isem1, gsem0, gsem1, ssem0, ssem1):
    c = lax.axis_index("c")
    s = lax.axis_index("s")
    pltpu.sync_copy(zeros_hbm, zero_v)
    lo = s * ACC_PT
    _zero_acc(zero_v, acc, lo, isem0)
    plsc.subcore_barrier()
    e0 = c * (EP // 2) + s * (EP // 32)
    _edge_loop(EP // 32 // CHUNK,
               lambda r: srcx.at[pl.ds(e0 + r, CHUNK)],
               lambda r: dstx.at[pl.ds(e0 + r, CHUNK)],
               table, (i_s0, i_s1), (i_d0, i_d1), (r0, r1), acc,
               (isem0, isem1), (gsem0, gsem1), (ssem0, ssem1))
    plsc.subcore_barrier()
    pltpu.sync_copy(acc.at[pl.ds(lo, ACC_PT)], out.at[c, pl.ds(lo, ACC_PT)])


def _sc_agg4_body(table, srcx, dstx, zeros_hbm, out,
                  i_s0, i_s1, i_d0, i_d1, r0, r1, zero_v, acc,
                  isem0, isem1, gsem0, gsem1, ssem0, ssem1):
    c = lax.axis_index("c")
    s = lax.axis_index("s")
    pltpu.sync_copy(zeros_hbm, zero_v)
    lo = s * ACC_PT
    e0 = s * (EP // 16)
    for rnd in range(2):
        g = 2 * rnd + c
        _zero_acc(zero_v, acc, lo, isem0)
        plsc.subcore_barrier()
        _edge_loop(EP // 16 // CHUNK,
                   lambda r: srcx.at[g, pl.ds(e0 + r, CHUNK)],
                   lambda r: dstx.at[pl.ds(e0 + r, CHUNK)],
                   table, (i_s0, i_s1), (i_d0, i_d1), (r0, r1), acc,
                   (isem0, isem1), (gsem0, gsem1), (ssem0, ssem1))
        plsc.subcore_barrier()
        pltpu.sync_copy(acc.at[pl.ds(lo, ACC_PT)],
                        out.at[pl.ds(lo, ACC_PT), pl.ds(g * 16, 16)])
        plsc.subcore_barrier()


_SC_SCRATCH = [
    pltpu.VMEM((CHUNK,), i32),
    pltpu.VMEM((CHUNK,), i32),
    pltpu.VMEM((CHUNK,), i32),
    pltpu.VMEM((CHUNK,), i32),
    pltpu.VMEM((CHUNK, 16), f32),
    pltpu.VMEM((CHUNK, 16), f32),
    pltpu.VMEM((128, 16), f32),
    pltpu.VMEM_SHARED((XP, 16), f32),
    pltpu.SemaphoreType.DMA,
    pltpu.SemaphoreType.DMA,
    pltpu.SemaphoreType.DMA,
    pltpu.SemaphoreType.DMA,
    pltpu.SemaphoreType.DMA,
    pltpu.SemaphoreType.DMA,
]


def _make_sc_kernels():
    mesh = plsc.VectorSubcoreMesh(core_axis_name="c", subcore_axis_name="s")
    params = pltpu.CompilerParams(use_tc_tiling_on_sc=False)
    agg1 = pl.kernel(
        _sc_agg1_body,
        out_type=jax.ShapeDtypeStruct((2, XP, 16), f32),
        mesh=mesh,
        scratch_types=_SC_SCRATCH,
        compiler_params=params,
        name="sc_gin_agg1",
    )
    agg4 = pl.kernel(
        _sc_agg4_body,
        out_type=jax.ShapeDtypeStruct((XP, HID), f32),
        mesh=mesh,
        scratch_types=_SC_SCRATCH,
        compiler_params=params,
        name="sc_gin_agg4",
    )
    return agg1, agg4


def _row_mask(b):
    rows = lax.broadcasted_iota(i32, (BLK, 1), 0) + b * BLK
    return rows < N


def _a1_body(x_ref, aa_ref, ab_ref, w1_ref, b1_ref, y_ref, st_ref):
    b = pl.program_id(0)
    u = x_ref[...] + aa_ref[0] + ab_ref[0]
    y = jnp.dot(u, w1_ref[...], preferred_element_type=f32, precision=lax.Precision.HIGHEST) + b1_ref[...]
    y_ref[...] = y
    @pl.when(b == 0)
    def _():
        st_ref[...] = jnp.zeros_like(st_ref)

    @pl.when(b < NBLK - 1)
    def _():
        st_ref[0:1, :] += jnp.sum(y, axis=0, keepdims=True)
        st_ref[1:2, :] += jnp.sum(y * y, axis=0, keepdims=True)

    @pl.when(b == NBLK - 1)
    def _():
        ym = jnp.where(_row_mask(b), y, 0.0)
        st_ref[0:1, :] += jnp.sum(ym, axis=0, keepdims=True)
        st_ref[1:2, :] += jnp.sum(ym * ym, axis=0, keepdims=True)


def _a4_body(h4_ref, agg_ref, w1_ref, b1_ref, y_ref, st_ref):
    b = pl.program_id(0)
    h = jnp.concatenate([h4_ref[g] for g in range(4)], axis=1)
    u = h + agg_ref[...]
    y = jnp.dot(u, w1_ref[...], preferred_element_type=f32,
                precision=lax.Precision.HIGHEST) + b1_ref[...]
    y_ref[...] = y
    @pl.when(b == 0)
    def _():
        st_ref[...] = jnp.zeros_like(st_ref)

    @pl.when(b < NBLK - 1)
    def _():
        st_ref[0:1, :] += jnp.sum(y, axis=0, keepdims=True)
        st_ref[1:2, :] += jnp.sum(y * y, axis=0, keepdims=True)

    @pl.when(b == NBLK - 1)
    def _():
        ym = jnp.where(_row_mask(b), y, 0.0)
        st_ref[0:1, :] += jnp.sum(ym, axis=0, keepdims=True)
        st_ref[1:2, :] += jnp.sum(ym * ym, axis=0, keepdims=True)


def _b_body(y_ref, st_ref, gamma_ref, beta_ref, w2_ref, b2_ref, h4_ref):
    inv_n = 1.0 / N
    mean = st_ref[0:1, :] * inv_n
    var = st_ref[1:2, :] * inv_n - mean * mean
    scale = gamma_ref[...] * lax.rsqrt(var + 1e-5)
    shift = beta_ref[...] - mean * scale
    h = jnp.maximum(y_ref[...] * scale + shift, 0.0)
    h = jnp.dot(h, w2_ref[...], preferred_element_type=f32, precision=lax.Precision.HIGHEST) + b2_ref[...]
    h = jnp.maximum(h, 0.0)
    for g in range(4):
        h4_ref[g] = h[:, g * 16:(g + 1) * 16]


def _pool_body(h4_ref, ids_ref, w1_ref, b1_ref, w2_ref, b2_ref, out_ref,
               sums, maxs, cnts):
    b = pl.program_id(0)

    @pl.when(b == 0)
    def _():
        sums[...] = jnp.zeros_like(sums)
        cnts[...] = jnp.zeros_like(cnts)
        maxs[...] = jnp.full_like(maxs, -1e30)

    ids = ids_ref[...]
    h = jnp.concatenate([h4_ref[g] for g in range(4)], axis=1)
    valid = ids >= 0
    gid = lax.broadcasted_iota(i32, (BLK, NG), 1)
    oh = jnp.where(valid & (ids == gid), 1.0, 0.0)
    ones_col = jnp.ones((BLK, 1), f32)
    dn = (((0,), (0,)), ((), ()))
    sums[...] += lax.dot_general(oh, h, dn, preferred_element_type=f32, precision=lax.Precision.HIGHEST)
    cnts[...] += lax.dot_general(oh, ones_col, dn, preferred_element_type=f32, precision=lax.Precision.HIGHEST)

    m = jnp.where(valid, h, -1e30)
    s = 1
    while s < BLK:
        ids_sh = jnp.concatenate(
            [ids[s:], jnp.full((s, 1), -2, i32)], axis=0)
        m_sh = jnp.concatenate(
            [m[s:], jnp.full((s, HID), -1e30, f32)], axis=0)
        m = jnp.where(ids_sh == ids, jnp.maximum(m, m_sh), m)
        s *= 2
    prev = jnp.concatenate([jnp.full((1, 1), -3, i32), ids[:-1]], axis=0)
    rs = jnp.where((ids != prev) & valid, 1.0, 0.0)
    ohm = oh * rs
    contrib = lax.dot_general(ohm, m, dn, preferred_element_type=f32, precision=lax.Precision.HIGHEST)
    pres = lax.dot_general(ohm, ones_col, dn, preferred_element_type=f32, precision=lax.Precision.HIGHEST)
    maxs[...] = jnp.maximum(maxs[...],
                            jnp.where(pres > 0, contrib, -1e30))

    @pl.when(b == NBLK - 1)
    def _():
        cnt = cnts[...]
        mean = sums[...] / jnp.maximum(cnt, 1.0)
        maxp = jnp.where(cnt > 0, maxs[...], 0.0)
        feat = jnp.concatenate([mean, maxp], axis=1)
        z = jnp.maximum(
            jnp.dot(feat, w1_ref[...], preferred_element_type=f32, precision=lax.Precision.HIGHEST)
            + b1_ref[...], 0.0)
        out_ref[...] = (jnp.dot(z, w2_ref[...], preferred_element_type=f32, precision=lax.Precision.HIGHEST)
                        + b2_ref[...])


_TC_PARAMS = pltpu.CompilerParams(dimension_semantics=("arbitrary",))


def _full(shape):
    return pl.BlockSpec(shape, lambda b: tuple(0 for _ in shape))


def _tc_a1(x, agg, w1p, b1):
    return pl.pallas_call(
        _a1_body,
        grid=(NBLK,),
        in_specs=[
            pl.BlockSpec((BLK, 16), lambda b: (b, 0)),
            pl.BlockSpec((1, BLK, 16), lambda b: (0, b, 0)),
            pl.BlockSpec((1, BLK, 16), lambda b: (1, b, 0)),
            _full((16, HID)),
            _full((1, HID)),
        ],
        out_specs=(pl.BlockSpec((BLK, HID), lambda b: (b, 0)),
                   _full((2, HID))),
        out_shape=(jax.ShapeDtypeStruct((XP, HID), f32),
                   jax.ShapeDtypeStruct((2, HID), f32)),
        compiler_params=_TC_PARAMS,
        name="tc_gin_a1",
    )(x, agg, agg, w1p, b1)


def _tc_a4(h4, agg, w1, b1):
    return pl.pallas_call(
        _a4_body,
        grid=(NBLK,),
        in_specs=[
            pl.BlockSpec((4, BLK, 16), lambda b: (0, b, 0)),
            pl.BlockSpec((BLK, HID), lambda b: (b, 0)),
            _full((HID, HID)),
            _full((1, HID)),
        ],
        out_specs=(pl.BlockSpec((BLK, HID), lambda b: (b, 0)),
                   _full((2, HID))),
        out_shape=(jax.ShapeDtypeStruct((XP, HID), f32),
                   jax.ShapeDtypeStruct((2, HID), f32)),
        compiler_params=_TC_PARAMS,
        name="tc_gin_a4",
    )(h4, agg, w1, b1)


def _tc_b(y, st, gamma, beta, w2, b2):
    return pl.pallas_call(
        _b_body,
        grid=(NBLK,),
        in_specs=[
            pl.BlockSpec((BLK, HID), lambda b: (b, 0)),
            _full((2, HID)),
            _full((1, HID)),
            _full((1, HID)),
            _full((HID, HID)),
            _full((1, HID)),
        ],
        out_specs=pl.BlockSpec((4, BLK, 16), lambda b: (0, b, 0)),
        out_shape=jax.ShapeDtypeStruct((4, XP, 16), f32),
        compiler_params=_TC_PARAMS,
        name="tc_gin_b",
    )(y, st, gamma, beta, w2, b2)


def _tc_pool(h4, ids, w1, b1, w2, b2):
    return pl.pallas_call(
        _pool_body,
        grid=(NBLK,),
        in_specs=[
            pl.BlockSpec((4, BLK, 16), lambda b: (0, b, 0)),
            pl.BlockSpec((BLK, 1), lambda b: (b, 0)),
            _full((2 * HID, HID)),
            _full((1, HID)),
            _full((HID, 2)),
            _full((1, 2)),
        ],
        out_specs=_full((NG, 2)),
        out_shape=jax.ShapeDtypeStruct((NG, 2), f32),
        scratch_shapes=[
            pltpu.VMEM((NG, HID), f32),
            pltpu.VMEM((NG, HID), f32),
            pltpu.VMEM((NG, 1), f32),
        ],
        compiler_params=_TC_PARAMS,
        name="tc_gin_pool",
    )(h4, ids, w1, b1, w2, b2)


def kernel(x, edge_index, batch,
           conv1_W1, conv1_b1, conv1_gamma, conv1_beta, conv1_W2, conv1_b2,
           conv2_W1, conv2_b1, conv2_gamma, conv2_beta, conv2_W2, conv2_b2,
           conv3_W1, conv3_b1, conv3_gamma, conv3_beta, conv3_W2, conv3_b2,
           clf_W1, clf_b1, clf_W2, clf_b2):
    sc_agg1, sc_agg4 = _make_sc_kernels()

    src = edge_index[0]
    dst = edge_index[1]
    srcp = jnp.pad(src, (0, EP - E))
    dstp = jnp.pad(dst, (0, EP - E), constant_values=DUMMY)
    srcx1 = srcp
    srcx4 = srcp[None, :] + (jnp.arange(4, dtype=i32) * XP)[:, None]
    dstx = dstp

    x_pad = jnp.pad(x, ((0, XP - N), (0, 16 - x.shape[1])))
    ids = jnp.pad(batch, (0, XP - N), constant_values=-1).reshape(XP, 1)
    w1p = jnp.pad(conv1_W1, ((0, 16 - conv1_W1.shape[0]), (0, 0)))

    def row(v):
        return v.reshape(1, -1)

    zeros_hbm = jnp.zeros((128, 16), f32)
    u1 = sc_agg1(x_pad, srcx1, dstx, zeros_hbm)
    y1, st1 = _tc_a1(x_pad, u1, w1p, row(conv1_b1))
    h1 = _tc_b(y1, st1, row(conv1_gamma), row(conv1_beta), conv1_W2,
               row(conv1_b2))

    a2 = sc_agg4(h1.reshape(4 * XP, 16), srcx4, dstx, zeros_hbm)
    y2, st2 = _tc_a4(h1, a2, conv2_W1, row(conv2_b1))
    h2 = _tc_b(y2, st2, row(conv2_gamma), row(conv2_beta), conv2_W2,
               row(conv2_b2))

    a3 = sc_agg4(h2.reshape(4 * XP, 16), srcx4, dstx, zeros_hbm)
    y3, st3 = _tc_a4(h2, a3, conv3_W1, row(conv3_b1))
    h3 = _tc_b(y3, st3, row(conv3_gamma), row(conv3_beta), conv3_W2,
               row(conv3_b2))

    return _tc_pool(h3, ids, clf_W1, row(clf_b1), clf_W2, row(clf_b2))
